# baseline (speedup 1.0000x reference)
import jax
import jax.numpy as jnp
from jax.experimental import pallas as pl

_N_EXPERTS = 64
_BLOCK_ROWS = 4096
_K_SPLIT = 2
_IDX_MASK = _N_EXPERTS - 1


def _router_body(x_ref, wt_ref, logits_ref, idx_ref, gates_ref):
    j = pl.program_id(1)
    part = jnp.dot(x_ref[...], wt_ref[...], preferred_element_type=jnp.float32)

    @pl.when(j == 0)
    def _init():
        logits_ref[...] = part

    @pl.when(j > 0)
    def _acc():
        logits_ref[...] += part

    @pl.when(j == _K_SPLIT - 1)
    def _finalize():
        l = logits_ref[...]
        bits = jax.lax.bitcast_convert_type(l, jnp.int32)
        s = jnp.where(bits < 0, bits ^ 0x7FFFFFFF, bits)
        iota = jax.lax.broadcasted_iota(jnp.int32, l.shape, 1)
        key = (s & ~_IDX_MASK) | (_IDX_MASK - iota)
        k1 = jnp.max(key, axis=-1, keepdims=True)
        key2 = jnp.where(key == k1, jnp.int32(-0x80000000), key)
        k2 = jnp.max(key2, axis=-1, keepdims=True)
        i1 = _IDX_MASK - (k1 & _IDX_MASK)
        i2 = _IDX_MASK - (k2 & _IDX_MASK)
        v1 = jax.lax.bitcast_convert_type(
            jnp.where(k1 < 0, k1 ^ 0x7FFFFFFF, k1), jnp.float32)
        v2 = jax.lax.bitcast_convert_type(
            jnp.where(k2 < 0, k2 ^ 0x7FFFFFFF, k2), jnp.float32)
        e = jnp.exp(v2 - v1)
        g0 = 1.0 / (1.0 + e)
        idx_ref[...] = jnp.concatenate([i1, i2], axis=1)
        gates_ref[...] = jnp.concatenate([g0, 1.0 - g0], axis=1)


@jax.jit
def kernel(x, W):
    rows, dim = x.shape
    n_experts = W.shape[0]
    wt = W.T
    kdim = dim // _K_SPLIT
    grid = (rows // _BLOCK_ROWS, _K_SPLIT)
    logits, idx, gates = pl.pallas_call(
        _router_body,
        grid=grid,
        in_specs=[
            pl.BlockSpec((_BLOCK_ROWS, kdim), lambda i, j: (i, j)),
            pl.BlockSpec((kdim, n_experts), lambda i, j: (j, 0)),
        ],
        out_specs=[
            pl.BlockSpec((_BLOCK_ROWS, n_experts), lambda i, j: (i, 0)),
            pl.BlockSpec((_BLOCK_ROWS, 2), lambda i, j: (i, 0)),
            pl.BlockSpec((_BLOCK_ROWS, 2), lambda i, j: (i, 0)),
        ],
        out_shape=[
            jax.ShapeDtypeStruct((rows, n_experts), jnp.float32),
            jax.ShapeDtypeStruct((rows, 2), jnp.int32),
            jax.ShapeDtypeStruct((rows, 2), jnp.float32),
        ],
    )(x, wt)
    return (idx, gates, logits)


# speedup vs baseline: 1.3129x; 1.3129x over previous
import jax
import jax.numpy as jnp
from jax.experimental import pallas as pl

_N_EXPERTS = 64
_BLOCK_ROWS = 4096
_IDX_MASK = _N_EXPERTS - 1


def _router_body(x_ref, wt_ref, logits_ref, idx_ref, gates_ref):
    l = jnp.dot(x_ref[...], wt_ref[...], preferred_element_type=jnp.float32)
    logits_ref[...] = l
    bits = jax.lax.bitcast_convert_type(l, jnp.int32)
    s = jnp.where(bits < 0, bits ^ 0x7FFFFFFF, bits)
    iota = jax.lax.broadcasted_iota(jnp.int32, l.shape, 1)
    key = (s & ~_IDX_MASK) | (_IDX_MASK - iota)
    k1 = jnp.max(key, axis=-1, keepdims=True)
    key2 = jnp.where(key == k1, jnp.int32(-0x80000000), key)
    k2 = jnp.max(key2, axis=-1, keepdims=True)
    kk = jnp.concatenate([k1, k2], axis=1)
    idx_ref[...] = _IDX_MASK - (kk & _IDX_MASK)
    vv = jax.lax.bitcast_convert_type(
        jnp.where(kk < 0, kk ^ 0x7FFFFFFF, kk), jnp.float32)
    d = vv - jnp.concatenate([vv[:, 1:2], vv[:, 0:1]], axis=1)
    gates_ref[...] = 1.0 / (1.0 + jnp.exp(-d))


@jax.jit
def kernel(x, W):
    rows, dim = x.shape
    n_experts = W.shape[0]
    wt = W.T
    grid = (rows // _BLOCK_ROWS,)
    logits, idx, gates = pl.pallas_call(
        _router_body,
        grid=grid,
        in_specs=[
            pl.BlockSpec((_BLOCK_ROWS, dim), lambda i: (i, 0)),
            pl.BlockSpec((dim, n_experts), lambda i: (0, 0)),
        ],
        out_specs=[
            pl.BlockSpec((_BLOCK_ROWS, n_experts), lambda i: (i, 0)),
            pl.BlockSpec((_BLOCK_ROWS, 2), lambda i: (i, 0)),
            pl.BlockSpec((_BLOCK_ROWS, 2), lambda i: (i, 0)),
        ],
        out_shape=[
            jax.ShapeDtypeStruct((rows, n_experts), jnp.float32),
            jax.ShapeDtypeStruct((rows, 2), jnp.int32),
            jax.ShapeDtypeStruct((rows, 2), jnp.float32),
        ],
    )(x, wt)
    return (idx, gates, logits)
